# baseline (speedup 1.0000x reference)
import functools

import jax
import jax.numpy as jnp
from jax import lax
from jax.experimental import pallas as pl
from jax.experimental.pallas import tpu as pltpu
from jax.experimental.pallas import tpu_sc as plsc

H = 128
NC = 2
NS = 16
NW = NC * NS
CK = 128
BIGIDX = 1 << 30


def _mesh():
  return plsc.VectorSubcoreMesh(
      core_axis_name="c", subcore_axis_name="s", num_cores=NC,
      num_subcores=NS)


NR = 30720


N0P = 10240
N1P = 20480


@functools.partial(jax.jit, static_argnums=(2,))
def _sc_gather(x01, idx_all, e0):
  et = idx_all.shape[0]
  nch0 = e0 // NS // CK
  nch1 = (et - e0) // NS // CK
  assert nch0 % 2 == 0 and nch1 % 2 == 0

  @functools.partial(
      pl.kernel,
      mesh=_mesh(),
      out_type=jax.ShapeDtypeStruct((et, H), jnp.float32),
      scratch_types=[
          [pltpu.VMEM((CK,), jnp.int32)] * 2,
          [pltpu.VMEM((CK, H), jnp.float32)] * 2,
          [pltpu.SemaphoreType.DMA] * 6,
      ],
  )
  def k(x01_hbm, idx_hbm, out_hbm, ib, gb, sem):
    c = lax.axis_index("c")
    t = lax.axis_index("s")

    nch = jnp.where(c == 0, nch0, nch1)
    ebase = jnp.where(c == 0, t * nch0, NS * nch0 + t * nch1) * CK

    def i_desc(ci, b):
      return pltpu.make_async_copy(
          idx_hbm.at[pl.ds(ebase + ci * CK, CK)], ib[b], sem[b])

    def g_desc(b):
      return pltpu.make_async_copy(x01_hbm.at[ib[b]], gb[b], sem[2 + b])

    def w_desc(ci, b):
      return pltpu.make_async_copy(
          gb[b], out_hbm.at[pl.ds(ebase + ci * CK, CK)], sem[4 + b])

    i_desc(0, 0).start()
    i_desc(1, 1).start()

    def body(g, _):
      for b in (0, 1):
        ci = 2 * g + b
        i_desc(ci, b).wait()

        @pl.when(ci >= 2)
        def _():
          w_desc(ci - 2, b).wait()

        g_desc(b).start()
        g_desc(b).wait()

        @pl.when(ci + 2 < nch)
        def _():
          i_desc(ci + 2, b).start()

        w_desc(ci, b).start()
      return ()

    lax.fori_loop(0, nch // 2, body, ())
    w_desc(nch - 2, 0).wait()
    w_desc(nch - 1, 1).wait()

  return k(x01, idx_all)


@functools.partial(jax.jit, static_argnums=(2, 3))
def _sc_scatter(msg, ridx, ep, nrec):
  nh = nrec // NC
  accr = ((nh + 64) + 1023) // 1024 * 1024
  rpt = accr // NS
  nzc = rpt // 64
  nch = ep // NS // CK
  ock = 200
  noc = nh // ock
  nopt = (noc + NS - 1) // NS

  assert nch % 2 == 0

  @functools.partial(
      pl.kernel,
      mesh=_mesh(),
      out_type=jax.ShapeDtypeStruct((nrec, H), jnp.float32),
      scratch_types=[
          pltpu.VMEM((64, H), jnp.float32),
          [pltpu.VMEM((CK,), jnp.int32)] * 4,
          [pltpu.VMEM((CK, H), jnp.float32)] * 2,
          pltpu.VMEM_SHARED((accr, H), jnp.float32),
          [pltpu.SemaphoreType.DMA] * 6,
      ],
  )
  def k(msg_hbm, ridx_hbm, out_hbm, zb_v, ib, mb, acc, sem):
    c = lax.axis_index("c")
    t = lax.axis_index("s")

    def zb(i, _):
      zb_v[i // 8, pl.ds((i % 8) * 16, 16)] = jnp.zeros((16,), jnp.float32)
      return ()
    lax.fori_loop(0, 64 * 8, zb, ())

    zbase = t * rpt
    def zc(i, _):
      pltpu.sync_copy(zb_v, acc.at[pl.ds(zbase + i * 64, 64)])
      return ()
    lax.fori_loop(0, nzc, zc, ())

    lo = c * nh
    ebase = t * nch * CK
    li = [ib[2], ib[3]]

    def i_desc(ci, b):
      return pltpu.make_async_copy(
          ridx_hbm.at[pl.ds(ebase + ci * CK, CK)], ib[b], sem[b])

    def m_desc(ci, b):
      return pltpu.make_async_copy(
          msg_hbm.at[pl.ds(ebase + ci * CK, CK)], mb[b], sem[2 + b])

    def s_desc(b):
      return pltpu.make_async_copy(mb[b], acc.at[li[b]], sem[4 + b])

    i_desc(0, 0).start()
    m_desc(0, 0).start()
    plsc.subcore_barrier()

    def body(g, _):
      for b in (0, 1):
        ci = 2 * g + b
        nb = 1 - b
        i_desc(ci, b).wait()
        m_desc(ci, b).wait()

        def fix(j, _):
          v = ib[b][pl.ds(j * 16, 16)]
          lv = v - lo
          ok = (lv >= 0) & (lv < nh)
          li[b][pl.ds(j * 16, 16)] = jnp.where(ok, lv, nh)
          return ()
        lax.fori_loop(0, CK // 16, fix, ())

        @pl.when(ci >= 1)
        def _():
          s_desc(nb).wait()

        pltpu.async_copy(mb[b], acc.at[li[b]], sem[4 + b], add=True)

        @pl.when(ci + 1 < nch)
        def _():
          i_desc(ci + 1, nb).start()
          m_desc(ci + 1, nb).start()
      return ()

    lax.fori_loop(0, nch // 2, body, ())
    s_desc(1).wait()
    plsc.subcore_barrier()

    def oc(i, _):
      ch = t + i * NS
      @pl.when(ch < noc)
      def _():
        pltpu.sync_copy(acc.at[pl.ds(ch * ock, ock)],
                        out_hbm.at[pl.ds(lo + ch * ock, ock)])
      return ()
    lax.fori_loop(0, nopt, oc, ())

  return k(msg, ridx)


def _edge_body(send_b, rec_b, invt_b, w1s_b, w1r_b, w1i_b, b1_b, w2_b, b2_b,
               ewt_b, eb_b, out_b):
  f32 = jnp.float32
  h = (jnp.dot(send_b[...].astype(jnp.bfloat16), w1s_b[...],
               preferred_element_type=f32)
       + jnp.dot(rec_b[...].astype(jnp.bfloat16), w1r_b[...],
                 preferred_element_type=f32)
       + lax.dot_general(invt_b[...], w1i_b[...],
                         (((0,), (0,)), ((), ())), preferred_element_type=f32)
       + b1_b[...])
  h = h * jax.nn.sigmoid(h)
  m = (jnp.dot(h.astype(jnp.bfloat16), w2_b[...],
               preferred_element_type=f32) + b2_b[...])
  m = m * jax.nn.sigmoid(m)
  wg = jax.nn.sigmoid(jnp.sum(m * ewt_b[...], axis=1, keepdims=True)
                      + eb_b[0])
  out_b[...] = m * wg


BE = 1024


@functools.partial(jax.jit, static_argnums=(10, 11, 12))
def _edge_mlp(gath, invt, w1s, w1r, w1i, b1, w2, b2, ewt, eb, soff, roff,
              ep):
  grid = (ep // BE,)
  wspec = pl.BlockSpec((H, H), lambda i: (0, 0))
  bspec = pl.BlockSpec((1, H), lambda i: (0, 0))
  return pl.pallas_call(
      _edge_body,
      grid=grid,
      in_specs=[
          pl.BlockSpec((BE, H), lambda i: (soff + i, 0)),
          pl.BlockSpec((BE, H), lambda i: (roff + i, 0)),
          pl.BlockSpec((8, BE), lambda i: (0, i)),
          wspec, wspec,
          pl.BlockSpec((8, H), lambda i: (0, 0)),
          bspec, wspec, bspec, bspec,
          pl.BlockSpec(memory_space=pltpu.SMEM),
      ],
      out_specs=pl.BlockSpec((BE, H), lambda i: (i, 0)),
      out_shape=jax.ShapeDtypeStruct((ep, H), jnp.float32),
  )(gath, gath, invt, w1s, w1r, w1i, b1, w2, b2, ewt, eb)


def _node0_body(x_b, m_b, a_b, bq_b, b1_b, w2_b, b2_b, sw_b, sb_b, out_b):
  f32 = jnp.float32
  h = (jnp.dot(x_b[...], a_b[...], preferred_element_type=f32)
       + jnp.dot(m_b[...], bq_b[...], preferred_element_type=f32)
       + b1_b[...])
  h = h * jax.nn.sigmoid(h)
  out_b[...] = (jnp.dot(h, w2_b[...], preferred_element_type=f32) + b2_b[...]
                + jnp.dot(x_b[...], sw_b[...], preferred_element_type=f32)
                + sb_b[...])


def _node1_body(x_b, m1_b, m2_b, a_b, bq_b, c_b, b1_b, w2_b, b2_b, sw_b,
                sb_b, out_b):
  f32 = jnp.float32
  h = (jnp.dot(x_b[...], a_b[...], preferred_element_type=f32)
       + jnp.dot(m1_b[...], bq_b[...], preferred_element_type=f32)
       + jnp.dot(m2_b[...], c_b[...], preferred_element_type=f32)
       + b1_b[...])
  h = h * jax.nn.sigmoid(h)
  out_b[...] = (jnp.dot(h, w2_b[...], preferred_element_type=f32) + b2_b[...]
                + jnp.dot(x_b[...], sw_b[...], preferred_element_type=f32)
                + sb_b[...])


@jax.jit
def _node0(x, m, a, bq, b1, w2, b2, sw, sb):
  n = x.shape[0]
  bn = 1000
  wspec = pl.BlockSpec((H, H), lambda i: (0, 0))
  bspec = pl.BlockSpec((1, H), lambda i: (0, 0))
  nspec = pl.BlockSpec((bn, H), lambda i: (i, 0))
  return pl.pallas_call(
      _node0_body,
      grid=(n // bn,),
      in_specs=[nspec, nspec, wspec, wspec, bspec, wspec, bspec, wspec,
                bspec],
      out_specs=nspec,
      out_shape=jax.ShapeDtypeStruct((n, H), jnp.float32),
  )(x, m, a, bq, b1, w2, b2, sw, sb)


@jax.jit
def _node1(x, m1, m2, a, bq, c, b1, w2, b2, sw, sb):
  n = x.shape[0]
  bn = 1000
  wspec = pl.BlockSpec((H, H), lambda i: (0, 0))
  bspec = pl.BlockSpec((1, H), lambda i: (0, 0))
  nspec = pl.BlockSpec((bn, H), lambda i: (i, 0))
  return pl.pallas_call(
      _node1_body,
      grid=(n // bn,),
      in_specs=[nspec, nspec, nspec, wspec, wspec, wspec, bspec, wspec,
                bspec, wspec, bspec],
      out_specs=nspec,
      out_shape=jax.ShapeDtypeStruct((n, H), jnp.float32),
  )(x, m1, m2, a, bq, c, b1, w2, b2, sw, sb)


def _pad_to(x, n, val):
  e = x.shape[0]
  if e == n:
    return x
  return jnp.concatenate(
      [x, jnp.full((n - e,) + x.shape[1:], val, x.dtype)], axis=0)


def _pad_ep(e):
  return -(-e // (NW * CK * 2)) * (NW * CK * 2)


def kernel(x0, x1, adj_0_0, adj_0_1, adj_1_1, inv_0_0, inv_0_1, inv_1_1,
           mw1_0_0, mb1_0_0, mw2_0_0, mb2_0_0, ew_0_0, eb_0_0,
           mw1_0_1, mb1_0_1, mw2_0_1, mb2_0_1, ew_0_1, eb_0_1,
           mw1_1_1, mb1_1_1, mw2_1_1, mb2_1_1, ew_1_1, eb_1_1,
           u0w1, u0b1, u0w2, u0b2, u1w1, u1b1, u1w2, u1b2, sw, sb):
  n0 = x0.shape[0]
  n1 = x1.shape[0]
  bf16 = jnp.bfloat16
  x01 = jnp.concatenate([_pad_to(x0, N0P, 0.0),
                         _pad_to(x1, N1P, 0.0)], axis=0)

  sets = []
  for adj, inv, w1, soffs, roffs, nrec in (
      (adj_0_0, inv_0_0, mw1_0_0, 0, 0, n0),
      (adj_0_1, inv_0_1, mw1_0_1, 0, N0P, n1),
      (adj_1_1, inv_1_1, mw1_1_1, N0P, N0P, n1)):
    e = adj.shape[1]
    ep = _pad_ep(e)
    ninv = inv.shape[1]
    sidx = _pad_to(adj[0].astype(jnp.int32) + soffs, ep, 0)
    gidx = _pad_to(adj[1].astype(jnp.int32) + roffs, ep, 0)
    ridx = _pad_to(adj[1].astype(jnp.int32), ep, BIGIDX)
    invt = jnp.zeros((8, ep), bf16).at[:ninv, :e].set(inv.T.astype(bf16))
    w1s = w1[:H].astype(bf16)
    w1r = w1[H:2 * H].astype(bf16)
    w1i = jnp.zeros((8, H), bf16).at[:ninv].set(w1[2 * H:].astype(bf16))
    sets.append(dict(ep=ep, sidx=sidx, gidx=gidx, ridx=ridx, invt=invt,
                     w1s=w1s, w1r=w1r, w1i=w1i, nrec=nrec))

  ep00, ep01, ep11 = sets[0]['ep'], sets[1]['ep'], sets[2]['ep']
  idx_all = jnp.concatenate([
      sets[0]['sidx'], sets[0]['gidx'], sets[1]['sidx'],
      sets[1]['gidx'], sets[2]['sidx'], sets[2]['gidx']])
  e0 = 2 * ep00 + ep01
  gath = _sc_gather(x01, idx_all, e0)

  offs = [(0, ep00 // BE),
          (2 * ep00 // BE, e0 // BE),
          ((e0 + ep01) // BE, (e0 + ep01 + ep11) // BE)]
  msgs = []
  for s, (soff, roff), (b1, w2, b2, ew, eb) in zip(sets, offs, (
      (mb1_0_0, mw2_0_0, mb2_0_0, ew_0_0, eb_0_0),
      (mb1_0_1, mw2_0_1, mb2_0_1, ew_0_1, eb_0_1),
      (mb1_1_1, mw2_1_1, mb2_1_1, ew_1_1, eb_1_1))):
    msgs.append(_edge_mlp(gath, s['invt'], s['w1s'], s['w1r'], s['w1i'],
                          b1.reshape(1, H), w2.astype(bf16),
                          b2.reshape(1, H), ew.reshape(1, H), eb, soff,
                          roff, s['ep']))

  m00 = _sc_scatter(msgs[0], sets[0]['ridx'], sets[0]['ep'], n0)
  m01 = _sc_scatter(msgs[1], sets[1]['ridx'], sets[1]['ep'], n1)
  m11 = _sc_scatter(msgs[2], sets[2]['ridx'], sets[2]['ep'], n1)

  out0 = _node0(x0, m00, u0w1[:H], u0w1[H:], u0b1.reshape(1, H), u0w2,
                u0b2.reshape(1, H), sw, sb.reshape(1, H))
  out1 = _node1(x1, m01, m11, u1w1[:H], u1w1[H:2 * H], u1w1[2 * H:],
                u1b1.reshape(1, H), u1w2, u1b2.reshape(1, H), sw,
                sb.reshape(1, H))
  return (out0, out1)


# speedup vs baseline: 1.0764x; 1.0764x over previous
import functools

import jax
import jax.numpy as jnp
from jax import lax
from jax.experimental import pallas as pl
from jax.experimental.pallas import tpu as pltpu
from jax.experimental.pallas import tpu_sc as plsc

H = 128
NC = 2
NS = 16
NW = NC * NS
CK = 128
BIGIDX = 1 << 30


def _mesh():
  return plsc.VectorSubcoreMesh(
      core_axis_name="c", subcore_axis_name="s", num_cores=NC,
      num_subcores=NS)


NR = 30720


N0P = 10240
N1P = 20480


@functools.partial(jax.jit, static_argnums=(2,))
def _sc_gather(x01, idx_all, e0):
  et = idx_all.shape[0]
  nch = et // NW // CK
  assert nch % 4 == 0

  @functools.partial(
      pl.kernel,
      mesh=_mesh(),
      out_type=jax.ShapeDtypeStruct((et, H), jnp.float32),
      scratch_types=[
          [pltpu.VMEM((CK,), jnp.int32)] * 4,
          [pltpu.VMEM((CK, H), jnp.float32)] * 4,
          [pltpu.SemaphoreType.DMA] * 12,
      ],
  )
  def k(x01_hbm, idx_hbm, out_hbm, ib, gb, sem):
    c = lax.axis_index("c")
    t = lax.axis_index("s")
    wid = t * NC + c
    ebase = wid * nch * CK

    def i_desc(ci, b):
      return pltpu.make_async_copy(
          idx_hbm.at[pl.ds(ebase + ci * CK, CK)], ib[b], sem[b])

    def g_desc(b):
      return pltpu.make_async_copy(x01_hbm.at[ib[b]], gb[b], sem[4 + b])

    def w_desc(ci, b):
      return pltpu.make_async_copy(
          gb[b], out_hbm.at[pl.ds(ebase + ci * CK, CK)], sem[8 + b])

    for j in range(4):
      i_desc(j, j).start()

    def body(g, _):
      for b in (0, 1, 2, 3):
        ci = 4 * g + b
        pb = (b + 2) % 4
        i_desc(ci, b).wait()

        @pl.when(ci >= 4)
        def _():
          w_desc(ci - 4, b).wait()

        g_desc(b).start()

        @pl.when(ci >= 2)
        def _():
          g_desc(pb).wait()
          w_desc(ci - 2, pb).start()

          @pl.when(ci + 2 < nch)
          def _():
            i_desc(ci + 2, pb).start()
      return ()

    lax.fori_loop(0, nch // 4, body, ())
    for ci in (nch - 2, nch - 1):
      b = ci % 4
      g_desc(b).wait()
      w_desc(ci, b).start()
    for ci in (nch - 4, nch - 3, nch - 2, nch - 1):
      w_desc(ci, ci % 4).wait()

  return k(x01, idx_all)


@functools.partial(jax.jit, static_argnums=(2, 3))
def _sc_scatter(msg, ridx, ep, nrec):
  nh = nrec // NC
  accr = ((nh + 64) + 1023) // 1024 * 1024
  rpt = accr // NS
  nzc = rpt // 64
  nch = ep // NS // CK
  ock = 200
  noc = nh // ock
  nopt = (noc + NS - 1) // NS

  assert nch % 2 == 0

  @functools.partial(
      pl.kernel,
      mesh=_mesh(),
      out_type=jax.ShapeDtypeStruct((nrec, H), jnp.float32),
      scratch_types=[
          pltpu.VMEM((64, H), jnp.float32),
          [pltpu.VMEM((CK,), jnp.int32)] * 4,
          [pltpu.VMEM((CK, H), jnp.float32)] * 2,
          pltpu.VMEM_SHARED((accr, H), jnp.float32),
          [pltpu.SemaphoreType.DMA] * 6,
      ],
  )
  def k(msg_hbm, ridx_hbm, out_hbm, zb_v, ib, mb, acc, sem):
    c = lax.axis_index("c")
    t = lax.axis_index("s")

    def zb(i, _):
      zb_v[i // 8, pl.ds((i % 8) * 16, 16)] = jnp.zeros((16,), jnp.float32)
      return ()
    lax.fori_loop(0, 64 * 8, zb, ())

    zbase = t * rpt
    def zc(i, _):
      pltpu.sync_copy(zb_v, acc.at[pl.ds(zbase + i * 64, 64)])
      return ()
    lax.fori_loop(0, nzc, zc, ())

    lo = c * nh
    ebase = t * nch * CK
    li = [ib[2], ib[3]]

    def i_desc(ci, b):
      return pltpu.make_async_copy(
          ridx_hbm.at[pl.ds(ebase + ci * CK, CK)], ib[b], sem[b])

    def m_desc(ci, b):
      return pltpu.make_async_copy(
          msg_hbm.at[pl.ds(ebase + ci * CK, CK)], mb[b], sem[2 + b])

    def s_desc(b):
      return pltpu.make_async_copy(mb[b], acc.at[li[b]], sem[4 + b])

    i_desc(0, 0).start()
    m_desc(0, 0).start()
    plsc.subcore_barrier()

    def body(g, _):
      for b in (0, 1):
        ci = 2 * g + b
        nb = 1 - b
        i_desc(ci, b).wait()
        m_desc(ci, b).wait()

        def fix(j, _):
          v = ib[b][pl.ds(j * 16, 16)]
          lv = v - lo
          ok = (lv >= 0) & (lv < nh)
          li[b][pl.ds(j * 16, 16)] = jnp.where(ok, lv, nh)
          return ()
        lax.fori_loop(0, CK // 16, fix, ())

        @pl.when(ci >= 1)
        def _():
          s_desc(nb).wait()

        pltpu.async_copy(mb[b], acc.at[li[b]], sem[4 + b], add=True)

        @pl.when(ci + 1 < nch)
        def _():
          i_desc(ci + 1, nb).start()
          m_desc(ci + 1, nb).start()
      return ()

    lax.fori_loop(0, nch // 2, body, ())
    s_desc(1).wait()
    plsc.subcore_barrier()

    def oc(i, _):
      ch = t + i * NS
      @pl.when(ch < noc)
      def _():
        pltpu.sync_copy(acc.at[pl.ds(ch * ock, ock)],
                        out_hbm.at[pl.ds(lo + ch * ock, ock)])
      return ()
    lax.fori_loop(0, nopt, oc, ())

  return k(msg, ridx)


def _edge_body(send_b, rec_b, invt_b, w1s_b, w1r_b, w1i_b, b1_b, w2_b, b2_b,
               ewt_b, eb_b, out_b):
  f32 = jnp.float32
  h = (jnp.dot(send_b[...].astype(jnp.bfloat16), w1s_b[...],
               preferred_element_type=f32)
       + jnp.dot(rec_b[...].astype(jnp.bfloat16), w1r_b[...],
                 preferred_element_type=f32)
       + lax.dot_general(invt_b[...], w1i_b[...],
                         (((0,), (0,)), ((), ())), preferred_element_type=f32)
       + b1_b[...])
  h = h * jax.nn.sigmoid(h)
  m = (jnp.dot(h.astype(jnp.bfloat16), w2_b[...],
               preferred_element_type=f32) + b2_b[...])
  m = m * jax.nn.sigmoid(m)
  wg = jax.nn.sigmoid(jnp.sum(m * ewt_b[...], axis=1, keepdims=True)
                      + eb_b[0])
  out_b[...] = m * wg


BE = 1024


@functools.partial(jax.jit, static_argnums=(10, 11, 12))
def _edge_mlp(gath, invt, w1s, w1r, w1i, b1, w2, b2, ewt, eb, soff, roff,
              ep):
  grid = (ep // BE,)
  wspec = pl.BlockSpec((H, H), lambda i: (0, 0))
  bspec = pl.BlockSpec((1, H), lambda i: (0, 0))
  return pl.pallas_call(
      _edge_body,
      grid=grid,
      in_specs=[
          pl.BlockSpec((BE, H), lambda i: (soff + i, 0)),
          pl.BlockSpec((BE, H), lambda i: (roff + i, 0)),
          pl.BlockSpec((8, BE), lambda i: (0, i)),
          wspec, wspec,
          pl.BlockSpec((8, H), lambda i: (0, 0)),
          bspec, wspec, bspec, bspec,
          pl.BlockSpec(memory_space=pltpu.SMEM),
      ],
      out_specs=pl.BlockSpec((BE, H), lambda i: (i, 0)),
      out_shape=jax.ShapeDtypeStruct((ep, H), jnp.float32),
  )(gath, gath, invt, w1s, w1r, w1i, b1, w2, b2, ewt, eb)


def _node0_body(x_b, m_b, a_b, bq_b, b1_b, w2_b, b2_b, sw_b, sb_b, out_b):
  f32 = jnp.float32
  h = (jnp.dot(x_b[...], a_b[...], preferred_element_type=f32)
       + jnp.dot(m_b[...], bq_b[...], preferred_element_type=f32)
       + b1_b[...])
  h = h * jax.nn.sigmoid(h)
  out_b[...] = (jnp.dot(h, w2_b[...], preferred_element_type=f32) + b2_b[...]
                + jnp.dot(x_b[...], sw_b[...], preferred_element_type=f32)
                + sb_b[...])


def _node1_body(x_b, m1_b, m2_b, a_b, bq_b, c_b, b1_b, w2_b, b2_b, sw_b,
                sb_b, out_b):
  f32 = jnp.float32
  h = (jnp.dot(x_b[...], a_b[...], preferred_element_type=f32)
       + jnp.dot(m1_b[...], bq_b[...], preferred_element_type=f32)
       + jnp.dot(m2_b[...], c_b[...], preferred_element_type=f32)
       + b1_b[...])
  h = h * jax.nn.sigmoid(h)
  out_b[...] = (jnp.dot(h, w2_b[...], preferred_element_type=f32) + b2_b[...]
                + jnp.dot(x_b[...], sw_b[...], preferred_element_type=f32)
                + sb_b[...])


@jax.jit
def _node0(x, m, a, bq, b1, w2, b2, sw, sb):
  n = x.shape[0]
  bn = 1000
  wspec = pl.BlockSpec((H, H), lambda i: (0, 0))
  bspec = pl.BlockSpec((1, H), lambda i: (0, 0))
  nspec = pl.BlockSpec((bn, H), lambda i: (i, 0))
  return pl.pallas_call(
      _node0_body,
      grid=(n // bn,),
      in_specs=[nspec, nspec, wspec, wspec, bspec, wspec, bspec, wspec,
                bspec],
      out_specs=nspec,
      out_shape=jax.ShapeDtypeStruct((n, H), jnp.float32),
  )(x, m, a, bq, b1, w2, b2, sw, sb)


@jax.jit
def _node1(x, m1, m2, a, bq, c, b1, w2, b2, sw, sb):
  n = x.shape[0]
  bn = 1000
  wspec = pl.BlockSpec((H, H), lambda i: (0, 0))
  bspec = pl.BlockSpec((1, H), lambda i: (0, 0))
  nspec = pl.BlockSpec((bn, H), lambda i: (i, 0))
  return pl.pallas_call(
      _node1_body,
      grid=(n // bn,),
      in_specs=[nspec, nspec, nspec, wspec, wspec, wspec, bspec, wspec,
                bspec, wspec, bspec],
      out_specs=nspec,
      out_shape=jax.ShapeDtypeStruct((n, H), jnp.float32),
  )(x, m1, m2, a, bq, c, b1, w2, b2, sw, sb)


def _pad_to(x, n, val):
  e = x.shape[0]
  if e == n:
    return x
  return jnp.concatenate(
      [x, jnp.full((n - e,) + x.shape[1:], val, x.dtype)], axis=0)


def _pad_ep(e):
  return -(-e // (NW * CK * 2)) * (NW * CK * 2)


def kernel(x0, x1, adj_0_0, adj_0_1, adj_1_1, inv_0_0, inv_0_1, inv_1_1,
           mw1_0_0, mb1_0_0, mw2_0_0, mb2_0_0, ew_0_0, eb_0_0,
           mw1_0_1, mb1_0_1, mw2_0_1, mb2_0_1, ew_0_1, eb_0_1,
           mw1_1_1, mb1_1_1, mw2_1_1, mb2_1_1, ew_1_1, eb_1_1,
           u0w1, u0b1, u0w2, u0b2, u1w1, u1b1, u1w2, u1b2, sw, sb):
  n0 = x0.shape[0]
  n1 = x1.shape[0]
  bf16 = jnp.bfloat16
  x01 = jnp.concatenate([_pad_to(x0, N0P, 0.0),
                         _pad_to(x1, N1P, 0.0)], axis=0)

  sets = []
  for adj, inv, w1, soffs, roffs, nrec in (
      (adj_0_0, inv_0_0, mw1_0_0, 0, 0, n0),
      (adj_0_1, inv_0_1, mw1_0_1, 0, N0P, n1),
      (adj_1_1, inv_1_1, mw1_1_1, N0P, N0P, n1)):
    e = adj.shape[1]
    ep = _pad_ep(e)
    ninv = inv.shape[1]
    sidx = _pad_to(adj[0].astype(jnp.int32) + soffs, ep, 0)
    gidx = _pad_to(adj[1].astype(jnp.int32) + roffs, ep, 0)
    ridx = _pad_to(adj[1].astype(jnp.int32), ep, BIGIDX)
    invt = jnp.zeros((8, ep), bf16).at[:ninv, :e].set(inv.T.astype(bf16))
    w1s = w1[:H].astype(bf16)
    w1r = w1[H:2 * H].astype(bf16)
    w1i = jnp.zeros((8, H), bf16).at[:ninv].set(w1[2 * H:].astype(bf16))
    sets.append(dict(ep=ep, sidx=sidx, gidx=gidx, ridx=ridx, invt=invt,
                     w1s=w1s, w1r=w1r, w1i=w1i, nrec=nrec))

  ep00, ep01, ep11 = sets[0]['ep'], sets[1]['ep'], sets[2]['ep']
  idx_all = jnp.concatenate([
      sets[0]['sidx'], sets[0]['gidx'], sets[1]['sidx'],
      sets[1]['gidx'], sets[2]['sidx'], sets[2]['gidx']])
  e0 = 2 * ep00 + ep01
  gath = _sc_gather(x01, idx_all, e0)

  offs = [(0, ep00 // BE),
          (2 * ep00 // BE, e0 // BE),
          ((e0 + ep01) // BE, (e0 + ep01 + ep11) // BE)]
  msgs = []
  for s, (soff, roff), (b1, w2, b2, ew, eb) in zip(sets, offs, (
      (mb1_0_0, mw2_0_0, mb2_0_0, ew_0_0, eb_0_0),
      (mb1_0_1, mw2_0_1, mb2_0_1, ew_0_1, eb_0_1),
      (mb1_1_1, mw2_1_1, mb2_1_1, ew_1_1, eb_1_1))):
    msgs.append(_edge_mlp(gath, s['invt'], s['w1s'], s['w1r'], s['w1i'],
                          b1.reshape(1, H), w2.astype(bf16),
                          b2.reshape(1, H), ew.reshape(1, H), eb, soff,
                          roff, s['ep']))

  m00 = _sc_scatter(msgs[0], sets[0]['ridx'], sets[0]['ep'], n0)
  m01 = _sc_scatter(msgs[1], sets[1]['ridx'], sets[1]['ep'], n1)
  m11 = _sc_scatter(msgs[2], sets[2]['ridx'], sets[2]['ep'], n1)

  out0 = _node0(x0, m00, u0w1[:H], u0w1[H:], u0b1.reshape(1, H), u0w2,
                u0b2.reshape(1, H), sw, sb.reshape(1, H))
  out1 = _node1(x1, m01, m11, u1w1[:H], u1w1[H:2 * H], u1w1[2 * H:],
                u1b1.reshape(1, H), u1w2, u1b2.reshape(1, H), sw,
                sb.reshape(1, H))
  return (out0, out1)
